# baseline (speedup 1.0000x reference)
import functools

import jax
import jax.numpy as jnp
from jax import lax
from jax.experimental import pallas as pl
from jax.experimental.pallas import tpu as pltpu
from jax.experimental.pallas import tpu_sc as plsc

_N = 10000
_E = 320000
_D = 128
_NP = 10240
_NW = 32
_EPW = 10240
_EP = _NW * _EPW
_CB = 64
_NB = _EPW // _CB
_RPT = _NP // 16
_BLK = 512
_NBLK = _NP // _BLK

_SRC = (0, 1, 0, 2, 1, 2, 0, 1, 2)
_DST = (1, 0, 2, 0, 2, 1, 0, 1, 2)

_mesh = plsc.VectorSubcoreMesh(core_axis_name="c", subcore_axis_name="s")


@functools.partial(
    pl.kernel,
    out_type=jax.ShapeDtypeStruct((9, 2, 2, _NP), jnp.float32),
    mesh=_mesh,
    scratch_types=[
        pltpu.VMEM((_EPW,), jnp.int32),
        pltpu.VMEM((_EPW,), jnp.int32),
        pltpu.VMEM((_NP,), jnp.float32),
        pltpu.VMEM((_NP,), jnp.float32),
        pltpu.VMEM((16, _RPT), jnp.float32),
        pltpu.VMEM((_RPT,), jnp.float32),
        pltpu.VMEM_SHARED((16, _NP), jnp.float32),
    ],
    compiler_params=pltpu.CompilerParams(needs_layout_passes=False),
)
def _deg(src_hbm, dst_hbm, out_hbm,
         src_v, dst_v, scnt_v, dcnt_v, red_v, stripe_v, part_sh):
    c = lax.axis_index("c")
    s = lax.axis_index("s")
    wid = c * 16 + s
    ones = jnp.ones((16,), jnp.float32)
    for r in range(9):
        pltpu.sync_copy(src_hbm.at[r, wid], src_v)
        pltpu.sync_copy(dst_hbm.at[r, wid], dst_v)

        def zero(i, carry):
            z = jnp.zeros((16,), jnp.float32)
            scnt_v[pl.ds(i * 16, 16)] = z
            dcnt_v[pl.ds(i * 16, 16)] = z
            return carry
        lax.fori_loop(0, _NP // 16, zero, 0)

        def count(i, carry):
            plsc.addupdate_scatter(scnt_v, [src_v[pl.ds(i * 16, 16)]], ones)
            plsc.addupdate_scatter(dcnt_v, [dst_v[pl.ds(i * 16, 16)]], ones)
            return carry
        lax.fori_loop(0, _EPW // 16, count, 0)

        for which, cnt_v in ((0, scnt_v), (1, dcnt_v)):
            pltpu.sync_copy(cnt_v, part_sh.at[s])
            plsc.subcore_barrier()
            for t in range(16):
                pltpu.sync_copy(part_sh.at[t, pl.ds(s * _RPT, _RPT)],
                                red_v.at[t])

            def red(j, carry):
                acc = jnp.zeros((16,), jnp.float32)
                for t in range(16):
                    acc = acc + red_v[t, pl.ds(j * 16, 16)]
                stripe_v[pl.ds(j * 16, 16)] = acc
                return carry
            lax.fori_loop(0, _RPT // 16, red, 0)
            pltpu.sync_copy(stripe_v,
                            out_hbm.at[r, c, which, pl.ds(s * _RPT, _RPT)])
            plsc.subcore_barrier()


_NSET = 4
_GLEAD = 2


def _agg_body(x0, x1, x2, x3, x4, x5, x6, x7, x8,
              src_hbm, dst_hbm, zeros_hbm, out_hbm,
              sidx_v, d0, d1, d2, d3, r0, r1, r2, r3,
              g0, g1, g2, g3, s0, s1, s2, s3, acc_sh):
    c = lax.axis_index("c")
    s = lax.axis_index("s")
    wid = c * 16 + s
    xs = (x0, x1, x2, x3, x4, x5, x6, x7, x8)
    didx = (d0, d1, d2, d3)
    rows = (r0, r1, r2, r3)
    gsem = (g0, g1, g2, g3)
    ssem = (s0, s1, s2, s3)

    def fetch(r, b, m):
        pltpu.make_async_copy(
            xs[r].at[sidx_v.at[pl.ds(b * _CB, _CB)]], rows[m],
            gsem[m]).start()
        pltpu.make_async_copy(
            dst_hbm.at[r, wid, b], didx[m], gsem[m]).start()

    def gwait(r, b, m):
        pltpu.make_async_copy(
            xs[r].at[sidx_v.at[pl.ds(b * _CB, _CB)]], rows[m],
            gsem[m]).wait()
        pltpu.make_async_copy(
            dst_hbm.at[r, wid, b], didx[m], gsem[m]).wait()

    def sstart(m):
        pltpu.make_async_copy(rows[m], acc_sh.at[didx[m]],
                              ssem[m]).start(add=True)

    def swait(m):
        pltpu.make_async_copy(rows[m], acc_sh.at[didx[m]], ssem[m]).wait()

    for r in range(9):
        pltpu.sync_copy(zeros_hbm, acc_sh.at[pl.ds(s * _RPT, _RPT)])
        pltpu.sync_copy(src_hbm.at[r, wid], sidx_v)
        plsc.subcore_barrier()
        for b0 in range(_GLEAD):
            fetch(r, b0, b0)

        def body(i, carry):
            for k in range(_NSET):
                b = i * _NSET + k
                gwait(r, b, k)
                sstart(k)
                if k >= _GLEAD:
                    swait(k - _GLEAD)

                    @pl.when(i < _NB // _NSET - 1)
                    def _():
                        fetch(r, b + _GLEAD, (k + _GLEAD) % _NSET)
                else:
                    @pl.when(i > 0)
                    def _():
                        swait((k - _GLEAD) % _NSET)
                    fetch(r, b + _GLEAD, k + _GLEAD)
            return carry

        lax.fori_loop(0, _NB // _NSET, body, 0)
        for m in range(_NSET - _GLEAD, _NSET):
            swait(m)
        plsc.subcore_barrier()
        pltpu.sync_copy(acc_sh.at[pl.ds(s * _RPT, _RPT)],
                        out_hbm.at[2 * r + c, pl.ds(s * _RPT, _RPT)])
        plsc.subcore_barrier()


_agg = functools.partial(
    pl.kernel,
    out_type=jax.ShapeDtypeStruct((18, _NP, _D), jnp.float32),
    mesh=_mesh,
    scratch_types=(
        [pltpu.VMEM((_EPW,), jnp.int32)]
        + [pltpu.VMEM((_CB,), jnp.int32)] * _NSET
        + [pltpu.VMEM((_CB, _D), jnp.float32)] * _NSET
        + [pltpu.SemaphoreType.DMA] * (2 * _NSET)
        + [pltpu.VMEM_SHARED((_NP, _D), jnp.float32)]
    ),
)(_agg_body)


def _inv_sqrt_deg(cnt_ref, r, which):
    deg = jnp.maximum(cnt_ref[r, 0, which] + cnt_ref[r, 1, which], 1.0)
    return lax.rsqrt(deg)[:, None]


def _prescale_body(f_ref, cnt_ref, *o_refs):
    for r in range(9):
        o_refs[r][...] = f_ref[_SRC[r]] * _inv_sqrt_deg(cnt_ref, r, 0)


def _prescale(feats3, cnts):
    return pl.pallas_call(
        _prescale_body,
        grid=(_NBLK,),
        in_specs=[
            pl.BlockSpec((3, _BLK, _D), lambda i: (0, i, 0)),
            pl.BlockSpec((9, 2, 2, _BLK), lambda i: (0, 0, 0, i)),
        ],
        out_specs=[pl.BlockSpec((_BLK, _D), lambda i: (i, 0))] * 9,
        out_shape=[jax.ShapeDtypeStruct((_NP, _D), jnp.float32)] * 9,
    )(feats3, cnts)


def _new_h(p_ref, cnt_ref, w_ref, b_ref):
    h = [jnp.zeros((_BLK, _D), jnp.float32) for _ in range(3)]
    for r in range(9):
        m = (p_ref[2 * r] + p_ref[2 * r + 1]) * _inv_sqrt_deg(cnt_ref, r, 1)
        h[_DST[r]] += (jnp.dot(m, w_ref[r], preferred_element_type=jnp.float32)
                       + b_ref[r][None, :])
    return h


def _combine1_body(p_ref, cnt_ref, w_ref, b_ref, *o_refs):
    h = _new_h(p_ref, cnt_ref, w_ref, b_ref)
    row = (pl.program_id(0) * _BLK
           + lax.broadcasted_iota(jnp.int32, (_BLK, 1), 0))
    mask = (row < _N).astype(jnp.float32)
    for d in range(3):
        h[d] = h[d] * mask
    for r in range(9):
        o_refs[r][...] = h[_SRC[r]] * _inv_sqrt_deg(cnt_ref, r, 0)


def _combine2_body(p_ref, cnt_ref, w_ref, b_ref, f0_ref, *o_refs):
    h = _new_h(p_ref, cnt_ref, w_ref, b_ref)
    for d in range(3):
        o_refs[d][...] = h[d] + f0_ref[d]


_P_SPEC = pl.BlockSpec((18, _BLK, _D), lambda i: (0, i, 0))
_CNT_SPEC = pl.BlockSpec((9, 2, 2, _BLK), lambda i: (0, 0, 0, i))
_W_SPEC = pl.BlockSpec((9, _D, _D), lambda i: (0, 0, 0))
_B_SPEC = pl.BlockSpec((9, _D), lambda i: (0, 0))
_O_SPEC = pl.BlockSpec((_BLK, _D), lambda i: (i, 0))


def _combine1(parts, cnts, w, b):
    return pl.pallas_call(
        _combine1_body,
        grid=(_NBLK,),
        in_specs=[_P_SPEC, _CNT_SPEC, _W_SPEC, _B_SPEC],
        out_specs=[_O_SPEC] * 9,
        out_shape=[jax.ShapeDtypeStruct((_NP, _D), jnp.float32)] * 9,
    )(parts, cnts, w, b)


def _combine2(parts, cnts, w, b, feats3):
    return pl.pallas_call(
        _combine2_body,
        grid=(_NBLK,),
        in_specs=[_P_SPEC, _CNT_SPEC, _W_SPEC, _B_SPEC,
                  pl.BlockSpec((3, _BLK, _D), lambda i: (0, i, 0))],
        out_specs=[_O_SPEC] * 3,
        out_shape=[jax.ShapeDtypeStruct((_NP, _D), jnp.float32)] * 3,
    )(parts, cnts, w, b, feats3)


def kernel(feat_a, feat_b, feat_g, ei_a2b, ei_b2a, ei_a2g, ei_g2a,
           ei_b2g, ei_g2b, ei_a2a, ei_b2b, ei_g2g, W, bias):
    eis = (ei_a2b, ei_b2a, ei_a2g, ei_g2a, ei_b2g, ei_g2b,
           ei_a2a, ei_b2b, ei_g2g)
    npad = _EP - _E
    pad = jnp.stack([
        _N + 128 + jnp.arange(npad, dtype=jnp.int32) % 112,
        _N + jnp.arange(npad, dtype=jnp.int32) % 128,
    ])
    ep = [jnp.concatenate([e, pad], axis=1) for e in eis]
    src = jnp.stack([e[0].reshape(_NW, _NB, _CB) for e in ep])
    dst = jnp.stack([e[1].reshape(_NW, _NB, _CB) for e in ep])
    feats3 = jnp.stack([
        jnp.pad(f, ((0, _NP - _N), (0, 0)))
        for f in (feat_a, feat_b, feat_g)
    ])
    zeros_d = jnp.zeros((_RPT, _D), jnp.float32)

    srcf = src.reshape(9, _NW, _EPW)
    cnts = _deg(srcf, dst.reshape(9, _NW, _EPW))
    xs = _prescale(feats3, cnts)
    p1 = _agg(*xs, srcf, dst, zeros_d)
    xs2 = _combine1(p1, cnts, W[0], bias[0])
    p2 = _agg(*xs2, srcf, dst, zeros_d)
    outs = _combine2(p2, cnts, W[1], bias[1], feats3)
    return outs[0][:_N], outs[1][:_N], outs[2][:_N]


# speedup vs baseline: 1.1031x; 1.1031x over previous
import functools

import jax
import jax.numpy as jnp
from jax import lax
from jax.experimental import pallas as pl
from jax.experimental.pallas import tpu as pltpu
from jax.experimental.pallas import tpu_sc as plsc

_N = 10000
_E = 320000
_D = 128
_NP = 10240
_NW = 32
_EPW = 10240
_EP = _NW * _EPW
_CB = 128
_NB = _EPW // _CB
_RPT = _NP // 16
_BLK = 512
_NBLK = _NP // _BLK

_SRC = (0, 1, 0, 2, 1, 2, 0, 1, 2)
_DST = (1, 0, 2, 0, 2, 1, 0, 1, 2)

_mesh = plsc.VectorSubcoreMesh(core_axis_name="c", subcore_axis_name="s")


@functools.partial(
    pl.kernel,
    out_type=jax.ShapeDtypeStruct((9, 2, 2, _NP), jnp.float32),
    mesh=_mesh,
    scratch_types=[
        pltpu.VMEM((_EPW,), jnp.int32),
        pltpu.VMEM((_EPW,), jnp.int32),
        pltpu.VMEM((_NP,), jnp.float32),
        pltpu.VMEM((_NP,), jnp.float32),
        pltpu.VMEM((16, _RPT), jnp.float32),
        pltpu.VMEM((_RPT,), jnp.float32),
        pltpu.VMEM_SHARED((16, _NP), jnp.float32),
    ],
    compiler_params=pltpu.CompilerParams(needs_layout_passes=False),
)
def _deg(src_hbm, dst_hbm, out_hbm,
         src_v, dst_v, scnt_v, dcnt_v, red_v, stripe_v, part_sh):
    c = lax.axis_index("c")
    s = lax.axis_index("s")
    wid = c * 16 + s
    ones = jnp.ones((16,), jnp.float32)
    for r in range(9):
        pltpu.sync_copy(src_hbm.at[r, wid], src_v)
        pltpu.sync_copy(dst_hbm.at[r, wid], dst_v)

        def zero(i, carry):
            z = jnp.zeros((16,), jnp.float32)
            scnt_v[pl.ds(i * 16, 16)] = z
            dcnt_v[pl.ds(i * 16, 16)] = z
            return carry
        lax.fori_loop(0, _NP // 16, zero, 0)

        def count(i, carry):
            plsc.addupdate_scatter(scnt_v, [src_v[pl.ds(i * 16, 16)]], ones)
            plsc.addupdate_scatter(dcnt_v, [dst_v[pl.ds(i * 16, 16)]], ones)
            return carry
        lax.fori_loop(0, _EPW // 16, count, 0)

        for which, cnt_v in ((0, scnt_v), (1, dcnt_v)):
            pltpu.sync_copy(cnt_v, part_sh.at[s])
            plsc.subcore_barrier()
            for t in range(16):
                pltpu.sync_copy(part_sh.at[t, pl.ds(s * _RPT, _RPT)],
                                red_v.at[t])

            def red(j, carry):
                acc = jnp.zeros((16,), jnp.float32)
                for t in range(16):
                    acc = acc + red_v[t, pl.ds(j * 16, 16)]
                stripe_v[pl.ds(j * 16, 16)] = acc
                return carry
            lax.fori_loop(0, _RPT // 16, red, 0)
            pltpu.sync_copy(stripe_v,
                            out_hbm.at[r, c, which, pl.ds(s * _RPT, _RPT)])
            plsc.subcore_barrier()


def _agg_body(x0, x1, x2, x3, x4, x5, x6, x7, x8,
              src_hbm, dst_hbm, zeros_hbm, out_hbm,
              sidx_v, didx_a, didx_b, rows_a, rows_b, sem_a, sem_b, acc_sh):
    c = lax.axis_index("c")
    s = lax.axis_index("s")
    wid = c * 16 + s
    xs = (x0, x1, x2, x3, x4, x5, x6, x7, x8)
    didx = (didx_a, didx_b)
    rows = (rows_a, rows_b)
    sems = (sem_a, sem_b)

    def fetch(r, b, slot):
        pltpu.make_async_copy(
            xs[r].at[sidx_v.at[pl.ds(b * _CB, _CB)]], rows[slot],
            sems[slot]).start()
        pltpu.make_async_copy(
            dst_hbm.at[r, wid, b], didx[slot], sems[slot]).start()

    def drain(r, b, slot):
        pltpu.make_async_copy(
            xs[r].at[sidx_v.at[pl.ds(b * _CB, _CB)]], rows[slot],
            sems[slot]).wait()
        pltpu.make_async_copy(
            dst_hbm.at[r, wid, b], didx[slot], sems[slot]).wait()

    for r in range(9):
        pltpu.sync_copy(zeros_hbm, acc_sh.at[pl.ds(s * _RPT, _RPT)])
        pltpu.sync_copy(src_hbm.at[r, wid], sidx_v)
        plsc.subcore_barrier()
        fetch(r, 0, 0)

        def body(i, carry):
            for k in range(2):
                b = i * 2 + k

                @pl.when(b + 1 < _NB)
                def _():
                    fetch(r, b + 1, 1 - k)

                drain(r, b, k)
                pltpu.sync_copy(rows[k], acc_sh.at[didx[k]], add=True)
            return carry

        lax.fori_loop(0, _NB // 2, body, 0)
        plsc.subcore_barrier()
        pltpu.sync_copy(acc_sh.at[pl.ds(s * _RPT, _RPT)],
                        out_hbm.at[2 * r + c, pl.ds(s * _RPT, _RPT)])
        plsc.subcore_barrier()


_agg = functools.partial(
    pl.kernel,
    out_type=jax.ShapeDtypeStruct((18, _NP, _D), jnp.float32),
    mesh=_mesh,
    scratch_types=[
        pltpu.VMEM((_EPW,), jnp.int32),
        pltpu.VMEM((_CB,), jnp.int32),
        pltpu.VMEM((_CB,), jnp.int32),
        pltpu.VMEM((_CB, _D), jnp.float32),
        pltpu.VMEM((_CB, _D), jnp.float32),
        pltpu.SemaphoreType.DMA,
        pltpu.SemaphoreType.DMA,
        pltpu.VMEM_SHARED((_NP, _D), jnp.float32),
    ],
)(_agg_body)


def _inv_sqrt_deg(cnt_ref, r, which):
    deg = jnp.maximum(cnt_ref[r, 0, which] + cnt_ref[r, 1, which], 1.0)
    return lax.rsqrt(deg)[:, None]


def _prescale_body(f_ref, cnt_ref, *o_refs):
    for r in range(9):
        o_refs[r][...] = f_ref[_SRC[r]] * _inv_sqrt_deg(cnt_ref, r, 0)


def _prescale(feats3, cnts):
    return pl.pallas_call(
        _prescale_body,
        grid=(_NBLK,),
        in_specs=[
            pl.BlockSpec((3, _BLK, _D), lambda i: (0, i, 0)),
            pl.BlockSpec((9, 2, 2, _BLK), lambda i: (0, 0, 0, i)),
        ],
        out_specs=[pl.BlockSpec((_BLK, _D), lambda i: (i, 0))] * 9,
        out_shape=[jax.ShapeDtypeStruct((_NP, _D), jnp.float32)] * 9,
    )(feats3, cnts)


def _new_h(p_ref, cnt_ref, w_ref, b_ref):
    h = [jnp.zeros((_BLK, _D), jnp.float32) for _ in range(3)]
    for r in range(9):
        m = (p_ref[2 * r] + p_ref[2 * r + 1]) * _inv_sqrt_deg(cnt_ref, r, 1)
        h[_DST[r]] += (jnp.dot(m, w_ref[r], preferred_element_type=jnp.float32)
                       + b_ref[r][None, :])
    return h


def _combine1_body(p_ref, cnt_ref, w_ref, b_ref, *o_refs):
    h = _new_h(p_ref, cnt_ref, w_ref, b_ref)
    row = (pl.program_id(0) * _BLK
           + lax.broadcasted_iota(jnp.int32, (_BLK, 1), 0))
    mask = (row < _N).astype(jnp.float32)
    for d in range(3):
        h[d] = h[d] * mask
    for r in range(9):
        o_refs[r][...] = h[_SRC[r]] * _inv_sqrt_deg(cnt_ref, r, 0)


def _combine2_body(p_ref, cnt_ref, w_ref, b_ref, f0_ref, *o_refs):
    h = _new_h(p_ref, cnt_ref, w_ref, b_ref)
    for d in range(3):
        o_refs[d][...] = h[d] + f0_ref[d]


_P_SPEC = pl.BlockSpec((18, _BLK, _D), lambda i: (0, i, 0))
_CNT_SPEC = pl.BlockSpec((9, 2, 2, _BLK), lambda i: (0, 0, 0, i))
_W_SPEC = pl.BlockSpec((9, _D, _D), lambda i: (0, 0, 0))
_B_SPEC = pl.BlockSpec((9, _D), lambda i: (0, 0))
_O_SPEC = pl.BlockSpec((_BLK, _D), lambda i: (i, 0))


def _combine1(parts, cnts, w, b):
    return pl.pallas_call(
        _combine1_body,
        grid=(_NBLK,),
        in_specs=[_P_SPEC, _CNT_SPEC, _W_SPEC, _B_SPEC],
        out_specs=[_O_SPEC] * 9,
        out_shape=[jax.ShapeDtypeStruct((_NP, _D), jnp.float32)] * 9,
    )(parts, cnts, w, b)


def _combine2(parts, cnts, w, b, feats3):
    return pl.pallas_call(
        _combine2_body,
        grid=(_NBLK,),
        in_specs=[_P_SPEC, _CNT_SPEC, _W_SPEC, _B_SPEC,
                  pl.BlockSpec((3, _BLK, _D), lambda i: (0, i, 0))],
        out_specs=[_O_SPEC] * 3,
        out_shape=[jax.ShapeDtypeStruct((_NP, _D), jnp.float32)] * 3,
    )(parts, cnts, w, b, feats3)


def kernel(feat_a, feat_b, feat_g, ei_a2b, ei_b2a, ei_a2g, ei_g2a,
           ei_b2g, ei_g2b, ei_a2a, ei_b2b, ei_g2g, W, bias):
    eis = (ei_a2b, ei_b2a, ei_a2g, ei_g2a, ei_b2g, ei_g2b,
           ei_a2a, ei_b2b, ei_g2g)
    npad = _EP - _E
    pad = jnp.stack([
        _N + 128 + jnp.arange(npad, dtype=jnp.int32) % 112,
        _N + jnp.arange(npad, dtype=jnp.int32) % 128,
    ])
    ep = [jnp.concatenate([e, pad], axis=1) for e in eis]
    src = jnp.stack([e[0].reshape(_NW, _NB, _CB) for e in ep])
    dst = jnp.stack([e[1].reshape(_NW, _NB, _CB) for e in ep])
    feats3 = jnp.stack([
        jnp.pad(f, ((0, _NP - _N), (0, 0)))
        for f in (feat_a, feat_b, feat_g)
    ])
    zeros_d = jnp.zeros((_RPT, _D), jnp.float32)

    srcf = src.reshape(9, _NW, _EPW)
    cnts = _deg(srcf, dst.reshape(9, _NW, _EPW))
    xs = _prescale(feats3, cnts)
    p1 = _agg(*xs, srcf, dst, zeros_d)
    xs2 = _combine1(p1, cnts, W[0], bias[0])
    p2 = _agg(*xs2, srcf, dst, zeros_d)
    outs = _combine2(p2, cnts, W[1], bias[1], feats3)
    return outs[0][:_N], outs[1][:_N], outs[2][:_N]


# speedup vs baseline: 1.1134x; 1.0093x over previous
import functools

import jax
import jax.numpy as jnp
from jax import lax
from jax.experimental import pallas as pl
from jax.experimental.pallas import tpu as pltpu
from jax.experimental.pallas import tpu_sc as plsc

_N = 10000
_E = 320000
_D = 128
_NP = 10240
_NW = 32
_EPW = 10240
_EP = _NW * _EPW
_CB = 128
_NB = _EPW // _CB
_RPT = _NP // 16
_BLK = 512
_NBLK = _NP // _BLK

_SRC = (0, 1, 0, 2, 1, 2, 0, 1, 2)
_DST = (1, 0, 2, 0, 2, 1, 0, 1, 2)

_mesh = plsc.VectorSubcoreMesh(core_axis_name="c", subcore_axis_name="s")


@functools.partial(
    pl.kernel,
    out_type=jax.ShapeDtypeStruct((9, 2, 2, _NP), jnp.float32),
    mesh=_mesh,
    scratch_types=[
        pltpu.VMEM((_EPW,), jnp.int32),
        pltpu.VMEM((_EPW,), jnp.int32),
        pltpu.VMEM((_NP,), jnp.float32),
        pltpu.VMEM((_NP,), jnp.float32),
        pltpu.VMEM((16, _RPT), jnp.float32),
        pltpu.VMEM((_RPT,), jnp.float32),
        pltpu.VMEM_SHARED((16, _NP), jnp.float32),
    ],
    compiler_params=pltpu.CompilerParams(needs_layout_passes=False),
)
def _deg(src_hbm, dst_hbm, out_hbm,
         src_v, dst_v, scnt_v, dcnt_v, red_v, stripe_v, part_sh):
    c = lax.axis_index("c")
    s = lax.axis_index("s")
    wid = c * 16 + s
    ones = jnp.ones((16,), jnp.float32)
    for r in range(9):
        pltpu.sync_copy(src_hbm.at[r, wid], src_v)
        pltpu.sync_copy(dst_hbm.at[r, wid], dst_v)

        def zero(i, carry):
            z = jnp.zeros((16,), jnp.float32)
            scnt_v[pl.ds(i * 16, 16)] = z
            dcnt_v[pl.ds(i * 16, 16)] = z
            return carry
        lax.fori_loop(0, _NP // 16, zero, 0)

        def count(i, carry):
            plsc.addupdate_scatter(scnt_v, [src_v[pl.ds(i * 16, 16)]], ones)
            plsc.addupdate_scatter(dcnt_v, [dst_v[pl.ds(i * 16, 16)]], ones)
            return carry
        lax.fori_loop(0, _EPW // 16, count, 0)

        for which, cnt_v in ((0, scnt_v), (1, dcnt_v)):
            pltpu.sync_copy(cnt_v, part_sh.at[s])
            plsc.subcore_barrier()
            for t in range(16):
                pltpu.sync_copy(part_sh.at[t, pl.ds(s * _RPT, _RPT)],
                                red_v.at[t])

            def red(j, carry):
                acc = jnp.zeros((16,), jnp.float32)
                for t in range(16):
                    acc = acc + red_v[t, pl.ds(j * 16, 16)]
                stripe_v[pl.ds(j * 16, 16)] = acc
                return carry
            lax.fori_loop(0, _RPT // 16, red, 0)
            pltpu.sync_copy(stripe_v,
                            out_hbm.at[r, c, which, pl.ds(s * _RPT, _RPT)])
            plsc.subcore_barrier()


def _agg_body(x0, x1, x2, x3, x4, x5, x6, x7, x8,
              src_hbm, dst_hbm, zeros_hbm, out_hbm,
              sidx_v, didx_a, didx_b, rows_a, rows_b, sem_a, sem_b, acc_sh):
    c = lax.axis_index("c")
    s = lax.axis_index("s")
    wid = c * 16 + s
    xs = (x0, x1, x2, x3, x4, x5, x6, x7, x8)
    didx = (didx_a, didx_b)
    rows = (rows_a, rows_b)
    sems = (sem_a, sem_b)

    def fetch(r, b, slot):
        pltpu.make_async_copy(
            xs[r].at[sidx_v.at[pl.ds(b * _CB, _CB)]], rows[slot],
            sems[slot]).start()
        pltpu.make_async_copy(
            dst_hbm.at[r, wid, pl.ds(b * _CB, _CB)], didx[slot],
            sems[slot]).start()

    def drain(r, b, slot):
        pltpu.make_async_copy(
            xs[r].at[sidx_v.at[pl.ds(b * _CB, _CB)]], rows[slot],
            sems[slot]).wait()
        pltpu.make_async_copy(
            dst_hbm.at[r, wid, pl.ds(b * _CB, _CB)], didx[slot],
            sems[slot]).wait()

    for r in range(9):
        pltpu.sync_copy(zeros_hbm, acc_sh.at[pl.ds(s * _RPT, _RPT)])
        pltpu.sync_copy(src_hbm.at[r, wid], sidx_v)
        plsc.subcore_barrier()
        fetch(r, 0, 0)

        def body(i, carry):
            for k in range(2):
                b = i * 2 + k

                @pl.when(b + 1 < _NB)
                def _():
                    fetch(r, b + 1, 1 - k)

                drain(r, b, k)
                pltpu.sync_copy(rows[k], acc_sh.at[didx[k]], add=True)
            return carry

        lax.fori_loop(0, _NB // 2, body, 0)
        plsc.subcore_barrier()
        pltpu.sync_copy(acc_sh.at[pl.ds(s * _RPT, _RPT)],
                        out_hbm.at[2 * r + c, pl.ds(s * _RPT, _RPT)])
        plsc.subcore_barrier()


_agg = functools.partial(
    pl.kernel,
    out_type=jax.ShapeDtypeStruct((18, _NP, _D), jnp.float32),
    mesh=_mesh,
    scratch_types=[
        pltpu.VMEM((_EPW,), jnp.int32),
        pltpu.VMEM((_CB,), jnp.int32),
        pltpu.VMEM((_CB,), jnp.int32),
        pltpu.VMEM((_CB, _D), jnp.float32),
        pltpu.VMEM((_CB, _D), jnp.float32),
        pltpu.SemaphoreType.DMA,
        pltpu.SemaphoreType.DMA,
        pltpu.VMEM_SHARED((_NP, _D), jnp.float32),
    ],
)(_agg_body)


def _inv_sqrt_deg(cnt_ref, r, which):
    deg = jnp.maximum(cnt_ref[r, 0, which] + cnt_ref[r, 1, which], 1.0)
    return lax.rsqrt(deg)[:, None]


def _prescale_body(f_ref, cnt_ref, *o_refs):
    for r in range(9):
        o_refs[r][...] = f_ref[_SRC[r]] * _inv_sqrt_deg(cnt_ref, r, 0)


def _prescale(feats3, cnts):
    return pl.pallas_call(
        _prescale_body,
        grid=(_NBLK,),
        in_specs=[
            pl.BlockSpec((3, _BLK, _D), lambda i: (0, i, 0)),
            pl.BlockSpec((9, 2, 2, _BLK), lambda i: (0, 0, 0, i)),
        ],
        out_specs=[pl.BlockSpec((_BLK, _D), lambda i: (i, 0))] * 9,
        out_shape=[jax.ShapeDtypeStruct((_NP, _D), jnp.float32)] * 9,
    )(feats3, cnts)


def _new_h(p_ref, cnt_ref, w_ref, b_ref):
    h = [jnp.zeros((_BLK, _D), jnp.float32) for _ in range(3)]
    for r in range(9):
        m = (p_ref[2 * r] + p_ref[2 * r + 1]) * _inv_sqrt_deg(cnt_ref, r, 1)
        h[_DST[r]] += (jnp.dot(m, w_ref[r], preferred_element_type=jnp.float32)
                       + b_ref[r][None, :])
    return h


def _combine1_body(p_ref, cnt_ref, w_ref, b_ref, *o_refs):
    h = _new_h(p_ref, cnt_ref, w_ref, b_ref)
    row = (pl.program_id(0) * _BLK
           + lax.broadcasted_iota(jnp.int32, (_BLK, 1), 0))
    mask = (row < _N).astype(jnp.float32)
    for d in range(3):
        h[d] = h[d] * mask
    for r in range(9):
        o_refs[r][...] = h[_SRC[r]] * _inv_sqrt_deg(cnt_ref, r, 0)


def _combine2_body(p_ref, cnt_ref, w_ref, b_ref, f0_ref, *o_refs):
    h = _new_h(p_ref, cnt_ref, w_ref, b_ref)
    for d in range(3):
        o_refs[d][...] = h[d] + f0_ref[d]


_P_SPEC = pl.BlockSpec((18, _BLK, _D), lambda i: (0, i, 0))
_CNT_SPEC = pl.BlockSpec((9, 2, 2, _BLK), lambda i: (0, 0, 0, i))
_W_SPEC = pl.BlockSpec((9, _D, _D), lambda i: (0, 0, 0))
_B_SPEC = pl.BlockSpec((9, _D), lambda i: (0, 0))
_O_SPEC = pl.BlockSpec((_BLK, _D), lambda i: (i, 0))


def _combine1(parts, cnts, w, b):
    return pl.pallas_call(
        _combine1_body,
        grid=(_NBLK,),
        in_specs=[_P_SPEC, _CNT_SPEC, _W_SPEC, _B_SPEC],
        out_specs=[_O_SPEC] * 9,
        out_shape=[jax.ShapeDtypeStruct((_NP, _D), jnp.float32)] * 9,
    )(parts, cnts, w, b)


def _combine2(parts, cnts, w, b, feats3):
    return pl.pallas_call(
        _combine2_body,
        grid=(_NBLK,),
        in_specs=[_P_SPEC, _CNT_SPEC, _W_SPEC, _B_SPEC,
                  pl.BlockSpec((3, _BLK, _D), lambda i: (0, i, 0))],
        out_specs=[_O_SPEC] * 3,
        out_shape=[jax.ShapeDtypeStruct((_NP, _D), jnp.float32)] * 3,
    )(parts, cnts, w, b, feats3)


def kernel(feat_a, feat_b, feat_g, ei_a2b, ei_b2a, ei_a2g, ei_g2a,
           ei_b2g, ei_g2b, ei_a2a, ei_b2b, ei_g2g, W, bias):
    eis = (ei_a2b, ei_b2a, ei_a2g, ei_g2a, ei_b2g, ei_g2b,
           ei_a2a, ei_b2b, ei_g2g)
    npad = _EP - _E
    pad = jnp.stack([
        _N + 128 + jnp.arange(npad, dtype=jnp.int32) % 112,
        _N + jnp.arange(npad, dtype=jnp.int32) % 128,
    ])
    ep = [jnp.concatenate([e, pad], axis=1) for e in eis]
    srcf = jnp.stack([e[0].reshape(_NW, _EPW) for e in ep])
    dstf = jnp.stack([e[1].reshape(_NW, _EPW) for e in ep])
    feats3 = jnp.stack([
        jnp.pad(f, ((0, _NP - _N), (0, 0)))
        for f in (feat_a, feat_b, feat_g)
    ])
    zeros_d = jnp.zeros((_RPT, _D), jnp.float32)

    cnts = _deg(srcf, dstf)
    xs = _prescale(feats3, cnts)
    p1 = _agg(*xs, srcf, dstf, zeros_d)
    xs2 = _combine1(p1, cnts, W[0], bias[0])
    p2 = _agg(*xs2, srcf, dstf, zeros_d)
    outs = _combine2(p2, cnts, W[1], bias[1], feats3)
    return outs[0][:_N], outs[1][:_N], outs[2][:_N]
